# Initial kernel scaffold; baseline (speedup 1.0000x reference)
import jax
import jax.numpy as jnp
from jax.experimental import pallas as pl


def kernel(x, edge_index, W1, b1, W2, b2):
    raise NotImplementedError("write your pallas kernel here")


# speedup vs baseline: 11.4458x; 11.4458x over previous
import functools

import jax
import jax.numpy as jnp
from jax import lax
from jax.experimental import pallas as pl
from jax.experimental.pallas import tpu as pltpu
from jax.experimental.pallas import tpu_sc as plsc

N = 10000
D = 128
E = 320000
NC = 2
NS = 16
NW = NC * NS

CH = 128

AGG_ROWS = E // CH
AGG_BASE = AGG_ROWS // NW
AGG_EXTRA = AGG_ROWS - AGG_BASE * NW

DEG_ROWS = (2 * E) // CH
DEG_BASE = DEG_ROWS // NW
DEG_EXTRA = DEG_ROWS - DEG_BASE * NW

ROWS_PER_TILE = N // NS
OC = 125
OCA = 25
DEG_W = 16
DROWS_PER_TILE = (2 * N) // NS


def _mesh():
    return plsc.VectorSubcoreMesh(core_axis_name="c", subcore_axis_name="s")


def _deg_body(idxr_hbm, upd_hbm, zero_hbm, out_hbm,
              idxbuf, ubuf, obuf, acc_sh, sem):
    c = lax.axis_index("c")
    s = lax.axis_index("s")
    w = c * NS + s

    pltpu.sync_copy(idxr_hbm.at[pl.ds(w * DEG_BASE, DEG_BASE)],
                    idxbuf.at[pl.ds(0, DEG_BASE)])

    @pl.when(w < DEG_EXTRA)
    def _():
        pltpu.sync_copy(idxr_hbm.at[pl.ds(NW * DEG_BASE + w, 1)],
                        idxbuf.at[pl.ds(DEG_BASE, 1)])

    pltpu.sync_copy(upd_hbm, ubuf)
    pltpu.sync_copy(zero_hbm, obuf)

    @pl.loop(0, DROWS_PER_TILE // OC)
    def _(k):
        pltpu.sync_copy(obuf, acc_sh.at[pl.ds(s * DROWS_PER_TILE + k * OC, OC)])

    plsc.subcore_barrier()

    @pl.loop(0, DEG_BASE)
    def _(j):
        pltpu.sync_copy(ubuf, acc_sh.at[idxbuf.at[j]], add=True)

    @pl.when(w < DEG_EXTRA)
    def _():
        pltpu.sync_copy(ubuf, acc_sh.at[idxbuf.at[DEG_BASE]], add=True)

    plsc.subcore_barrier()

    @pl.loop(0, DROWS_PER_TILE // OC)
    def _(k):
        base = s * DROWS_PER_TILE + k * OC
        pltpu.sync_copy(acc_sh.at[pl.ds(base, OC)], obuf)
        pltpu.sync_copy(obuf, out_hbm.at[pl.ds(c * 2 * N + base, OC)])


_sc_params = pltpu.CompilerParams(use_tc_tiling_on_sc=False)

_deg_call = pl.kernel(
    _deg_body,
    out_type=jax.ShapeDtypeStruct((2 * 2 * N, DEG_W), jnp.float32),
    mesh=_mesh(),
    compiler_params=_sc_params,
    scratch_types=[
        pltpu.VMEM((DEG_BASE + 1, CH), jnp.int32),
        pltpu.VMEM((CH, DEG_W), jnp.float32),
        pltpu.VMEM((OC, DEG_W), jnp.float32),
        pltpu.VMEM_SHARED((2 * N, DEG_W), jnp.float32),
        pltpu.SemaphoreType.DMA,
    ],
)


def _agg_body(g_hbm, srcr_hbm, dstr_hbm, zero_hbm, out_hbm,
              sidx, didx, rows0, obuf, acc_sh, sem0):
    c = lax.axis_index("c")
    s = lax.axis_index("s")
    w = c * NS + s

    pltpu.sync_copy(srcr_hbm.at[pl.ds(w * AGG_BASE, AGG_BASE)],
                    sidx.at[pl.ds(0, AGG_BASE)])
    pltpu.sync_copy(dstr_hbm.at[pl.ds(w * AGG_BASE, AGG_BASE)],
                    didx.at[pl.ds(0, AGG_BASE)])

    @pl.when(w < AGG_EXTRA)
    def _():
        pltpu.sync_copy(srcr_hbm.at[pl.ds(NW * AGG_BASE + w, 1)],
                        sidx.at[pl.ds(AGG_BASE, 1)])
        pltpu.sync_copy(dstr_hbm.at[pl.ds(NW * AGG_BASE + w, 1)],
                        didx.at[pl.ds(AGG_BASE, 1)])

    pltpu.sync_copy(zero_hbm, obuf)

    @pl.loop(0, ROWS_PER_TILE // OCA)
    def _(k):
        pltpu.sync_copy(obuf, acc_sh.at[pl.ds(s * ROWS_PER_TILE + k * OCA, OCA)])

    plsc.subcore_barrier()

    @pl.loop(0, AGG_BASE)
    def _(j):
        pltpu.async_copy(g_hbm.at[sidx.at[j]], rows0, sem0).wait()
        pltpu.sync_copy(rows0, acc_sh.at[didx.at[j]], add=True)

    @pl.when(w < AGG_EXTRA)
    def _():
        pltpu.async_copy(g_hbm.at[sidx.at[AGG_BASE]], rows0, sem0).wait()
        pltpu.sync_copy(rows0, acc_sh.at[didx.at[AGG_BASE]], add=True)

    plsc.subcore_barrier()

    @pl.loop(0, ROWS_PER_TILE // OCA)
    def _(k):
        base = s * ROWS_PER_TILE + k * OCA
        pltpu.sync_copy(acc_sh.at[pl.ds(base, OCA)], obuf)
        pltpu.sync_copy(obuf, out_hbm.at[pl.ds(c * N + base, OCA)])


_agg_call = pl.kernel(
    _agg_body,
    out_type=jax.ShapeDtypeStruct((2 * N, D), jnp.float32),
    mesh=_mesh(),
    compiler_params=_sc_params,
    scratch_types=[
        pltpu.VMEM((AGG_BASE + 1, CH), jnp.int32),
        pltpu.VMEM((AGG_BASE + 1, CH), jnp.int32),
        pltpu.VMEM((CH, D), jnp.float32),
        pltpu.VMEM((OCA, D), jnp.float32),
        pltpu.VMEM_SHARED((N, D), jnp.float32),
        pltpu.SemaphoreType.DMA,
    ],
)


RB = 1000


def _tc1_body(x_ref, ns_ref, w_ref, o_ref):
    o_ref[...] = jnp.dot(x_ref[...] * ns_ref[...], w_ref[...],
                         preferred_element_type=jnp.float32)


def _tc2_body(p_ref, g_ref, nd_ref, b_ref, ns_ref, w_ref, o_ref):
    h = (p_ref[0] + p_ref[1] + g_ref[...]) * nd_ref[...] + b_ref[...]
    h = jnp.maximum(h, 0.0)
    o_ref[...] = jnp.dot(h * ns_ref[...], w_ref[...],
                         preferred_element_type=jnp.float32)


def _tc3_body(p_ref, g_ref, nd_ref, b_ref, o_ref):
    o_ref[...] = (p_ref[0] + p_ref[1] + g_ref[...]) * nd_ref[...] + b_ref[...]


_vec_spec = pl.BlockSpec((RB, 1), lambda i: (i, 0))
_mat_spec = pl.BlockSpec((RB, D), lambda i: (i, 0))
_w_spec = pl.BlockSpec((D, D), lambda i: (0, 0))
_b_spec = pl.BlockSpec((1, D), lambda i: (0, 0))
_p_spec = pl.BlockSpec((2, RB, D), lambda i: (0, i, 0))

_tc1 = pl.pallas_call(
    _tc1_body,
    out_shape=jax.ShapeDtypeStruct((N, D), jnp.float32),
    grid=(N // RB,),
    in_specs=[_mat_spec, _vec_spec, _w_spec],
    out_specs=_mat_spec,
)

_tc2 = pl.pallas_call(
    _tc2_body,
    out_shape=jax.ShapeDtypeStruct((N, D), jnp.float32),
    grid=(N // RB,),
    in_specs=[_p_spec, _mat_spec, _vec_spec, _b_spec, _vec_spec, _w_spec],
    out_specs=_mat_spec,
)

_tc3 = pl.pallas_call(
    _tc3_body,
    out_shape=jax.ShapeDtypeStruct((N, D), jnp.float32),
    grid=(N // RB,),
    in_specs=[_p_spec, _mat_spec, _vec_spec, _b_spec],
    out_specs=_mat_spec,
)


def kernel(x, edge_index, W1, b1, W2, b2):
    src = edge_index[0]
    dst = edge_index[1]
    srcr = src.reshape(AGG_ROWS, CH)
    dstr = dst.reshape(AGG_ROWS, CH)
    idx2 = jnp.concatenate([src, dst + N]).reshape(DEG_ROWS, CH)

    upd = jnp.zeros((CH, DEG_W), jnp.float32).at[:, 0].set(1.0)
    zero16 = jnp.zeros((OC, DEG_W), jnp.float32)
    zero128 = jnp.zeros((OCA, D), jnp.float32)

    degp = _deg_call(idx2, upd, zero16)
    deg = degp[: 2 * N, 0] + degp[2 * N :, 0] + 1.0
    ns = lax.rsqrt(deg[:N])[:, None]
    nd = lax.rsqrt(deg[N:])[:, None]

    g1 = _tc1(x, ns, W1)
    p1 = _agg_call(g1, srcr, dstr, zero128)
    g2 = _tc2(p1.reshape(2, N, D), g1, nd, b1.reshape(1, D), ns, W2)
    p2 = _agg_call(g2, srcr, dstr, zero128)
    out = _tc3(p2.reshape(2, N, D), g2, nd, b2.reshape(1, D))
    return out


# speedup vs baseline: 15.6150x; 1.3643x over previous
import functools

import jax
import jax.numpy as jnp
from jax import lax
from jax.experimental import pallas as pl
from jax.experimental.pallas import tpu as pltpu
from jax.experimental.pallas import tpu_sc as plsc

N = 10000
D = 128
E = 320000
NC = 2
NS = 16
NW = NC * NS

CH = 128

AGG_ROWS = E // CH
AGG_BASE = AGG_ROWS // NW
AGG_EXTRA = AGG_ROWS - AGG_BASE * NW

DEG_ROWS = (2 * E) // CH
DEG_BASE = DEG_ROWS // NW
DEG_EXTRA = DEG_ROWS - DEG_BASE * NW

ROWS_PER_TILE = N // NS
OC = 125
OCA = 25
DEG_W = 16
DROWS_PER_TILE = (2 * N) // NS


def _mesh():
    return plsc.VectorSubcoreMesh(core_axis_name="c", subcore_axis_name="s")


def _deg_body(idxr_hbm, upd_hbm, zero_hbm, out_hbm,
              idxbuf, ubuf, obuf, acc_sh, sem):
    c = lax.axis_index("c")
    s = lax.axis_index("s")
    w = c * NS + s

    pltpu.sync_copy(idxr_hbm.at[pl.ds(w * DEG_BASE, DEG_BASE)],
                    idxbuf.at[pl.ds(0, DEG_BASE)])

    @pl.when(w < DEG_EXTRA)
    def _():
        pltpu.sync_copy(idxr_hbm.at[pl.ds(NW * DEG_BASE + w, 1)],
                        idxbuf.at[pl.ds(DEG_BASE, 1)])

    pltpu.sync_copy(upd_hbm, ubuf)
    pltpu.sync_copy(zero_hbm, obuf)

    @pl.loop(0, DROWS_PER_TILE // OC)
    def _(k):
        pltpu.sync_copy(obuf, acc_sh.at[pl.ds(s * DROWS_PER_TILE + k * OC, OC)])

    plsc.subcore_barrier()

    @pl.loop(0, DEG_BASE // 4)
    def _(jj):
        j = jj * 4
        pltpu.async_copy(ubuf, acc_sh.at[idxbuf.at[j]], sem, add=True)
        pltpu.async_copy(ubuf, acc_sh.at[idxbuf.at[j + 1]], sem, add=True)
        pltpu.async_copy(ubuf, acc_sh.at[idxbuf.at[j + 2]], sem, add=True)
        pltpu.async_copy(ubuf, acc_sh.at[idxbuf.at[j + 3]], sem, add=True)
        pltpu.make_async_copy(ubuf, acc_sh.at[idxbuf.at[j]], sem).wait()
        pltpu.make_async_copy(ubuf, acc_sh.at[idxbuf.at[j + 1]], sem).wait()
        pltpu.make_async_copy(ubuf, acc_sh.at[idxbuf.at[j + 2]], sem).wait()
        pltpu.make_async_copy(ubuf, acc_sh.at[idxbuf.at[j + 3]], sem).wait()

    @pl.when(w < DEG_EXTRA)
    def _():
        pltpu.sync_copy(ubuf, acc_sh.at[idxbuf.at[DEG_BASE]], add=True)

    plsc.subcore_barrier()

    @pl.loop(0, DROWS_PER_TILE // OC)
    def _(k):
        base = s * DROWS_PER_TILE + k * OC
        pltpu.sync_copy(acc_sh.at[pl.ds(base, OC)], obuf)
        pltpu.sync_copy(obuf, out_hbm.at[pl.ds(c * 2 * N + base, OC)])


_sc_params = pltpu.CompilerParams(use_tc_tiling_on_sc=False)

_deg_call = pl.kernel(
    _deg_body,
    out_type=jax.ShapeDtypeStruct((2 * 2 * N, DEG_W), jnp.float32),
    mesh=_mesh(),
    compiler_params=_sc_params,
    scratch_types=[
        pltpu.VMEM((DEG_BASE + 1, CH), jnp.int32),
        pltpu.VMEM((CH, DEG_W), jnp.float32),
        pltpu.VMEM((OC, DEG_W), jnp.float32),
        pltpu.VMEM_SHARED((2 * N, DEG_W), jnp.float32),
        pltpu.SemaphoreType.DMA,
    ],
)


BLK = 26
NBLK = AGG_BASE // BLK


def _agg_body(g_hbm, srcr_hbm, dstr_hbm, zero_hbm, out_hbm,
              sidx, didx, rows0, rows1, acc_sh, sem_g0, sem_g1, sem_i):
    c = lax.axis_index("c")
    s = lax.axis_index("s")
    w = c * NS + s

    pltpu.sync_copy(zero_hbm, rows0)

    @pl.loop(0, 4)
    def _(k):
        pltpu.sync_copy(rows0, acc_sh.at[pl.ds(s * ROWS_PER_TILE + k * CH, CH)])

    pltpu.sync_copy(rows0.at[pl.ds(0, ROWS_PER_TILE - 4 * CH)],
                    acc_sh.at[pl.ds(s * ROWS_PER_TILE + 4 * CH,
                                    ROWS_PER_TILE - 4 * CH)])
    plsc.subcore_barrier()

    pltpu.sync_copy(srcr_hbm.at[pl.ds(w * AGG_BASE, BLK)], sidx.at[pl.ds(0, BLK)])
    pltpu.sync_copy(dstr_hbm.at[pl.ds(w * AGG_BASE, BLK)], didx.at[pl.ds(0, BLK)])
    pltpu.async_copy(g_hbm.at[sidx.at[0]], rows0, sem_g0)

    @pl.loop(0, NBLK)
    def _(kk):
        roff = (kk % 2) * BLK
        noff = ((kk + 1) % 2) * BLK

        @pl.when(kk + 1 < NBLK)
        def _():
            pltpu.async_copy(srcr_hbm.at[pl.ds(w * AGG_BASE + (kk + 1) * BLK, BLK)],
                             sidx.at[pl.ds(noff, BLK)], sem_i)
            pltpu.async_copy(dstr_hbm.at[pl.ds(w * AGG_BASE + (kk + 1) * BLK, BLK)],
                             didx.at[pl.ds(noff, BLK)], sem_i)

        @pl.loop(0, BLK // 2)
        def _(pp):
            j0 = roff + 2 * pp
            j1 = j0 + 1
            pltpu.make_async_copy(g_hbm.at[sidx.at[j0]], rows0, sem_g0).wait()
            pltpu.async_copy(g_hbm.at[sidx.at[j1]], rows1, sem_g1)
            pltpu.sync_copy(rows0, acc_sh.at[didx.at[j0]], add=True)

            @pl.when(2 * pp + 2 < BLK)
            def _():
                pltpu.async_copy(g_hbm.at[sidx.at[j0 + 2]], rows0, sem_g0)

            pltpu.make_async_copy(g_hbm.at[sidx.at[j1]], rows1, sem_g1).wait()
            pltpu.sync_copy(rows1, acc_sh.at[didx.at[j1]], add=True)

        @pl.when(kk + 1 < NBLK)
        def _():
            pltpu.make_async_copy(srcr_hbm.at[pl.ds(0, BLK)],
                                  sidx.at[pl.ds(noff, BLK)], sem_i).wait()
            pltpu.make_async_copy(dstr_hbm.at[pl.ds(0, BLK)],
                                  didx.at[pl.ds(noff, BLK)], sem_i).wait()
            pltpu.async_copy(g_hbm.at[sidx.at[noff]], rows0, sem_g0)

    @pl.when(w < AGG_EXTRA)
    def _():
        pltpu.sync_copy(srcr_hbm.at[pl.ds(NW * AGG_BASE + w, 1)],
                        sidx.at[pl.ds(0, 1)])
        pltpu.sync_copy(dstr_hbm.at[pl.ds(NW * AGG_BASE + w, 1)],
                        didx.at[pl.ds(0, 1)])
        pltpu.async_copy(g_hbm.at[sidx.at[0]], rows0, sem_g0).wait()
        pltpu.sync_copy(rows0, acc_sh.at[didx.at[0]], add=True)

    plsc.subcore_barrier()

    @pl.loop(0, 4)
    def _(k):
        base = s * ROWS_PER_TILE + k * CH
        pltpu.sync_copy(acc_sh.at[pl.ds(base, CH)], rows0)
        pltpu.sync_copy(rows0, out_hbm.at[pl.ds(c * N + base, CH)])

    tail = ROWS_PER_TILE - 4 * CH
    base = s * ROWS_PER_TILE + 4 * CH
    pltpu.sync_copy(acc_sh.at[pl.ds(base, tail)], rows0.at[pl.ds(0, tail)])
    pltpu.sync_copy(rows0.at[pl.ds(0, tail)], out_hbm.at[pl.ds(c * N + base, tail)])


_agg_call = pl.kernel(
    _agg_body,
    out_type=jax.ShapeDtypeStruct((2 * N, D), jnp.float32),
    mesh=_mesh(),
    compiler_params=_sc_params,
    scratch_types=[
        pltpu.VMEM((2 * BLK, CH), jnp.int32),
        pltpu.VMEM((2 * BLK, CH), jnp.int32),
        pltpu.VMEM((CH, D), jnp.float32),
        pltpu.VMEM((CH, D), jnp.float32),
        pltpu.VMEM_SHARED((N, D), jnp.float32),
        pltpu.SemaphoreType.DMA,
        pltpu.SemaphoreType.DMA,
        pltpu.SemaphoreType.DMA,
    ],
)


RB = 1000


def _tc1_body(x_ref, ns_ref, w_ref, o_ref):
    o_ref[...] = jnp.dot(x_ref[...] * ns_ref[...], w_ref[...],
                         preferred_element_type=jnp.float32)


def _tc2_body(p_ref, g_ref, nd_ref, b_ref, ns_ref, w_ref, o_ref):
    h = (p_ref[0] + p_ref[1] + g_ref[...]) * nd_ref[...] + b_ref[...]
    h = jnp.maximum(h, 0.0)
    o_ref[...] = jnp.dot(h * ns_ref[...], w_ref[...],
                         preferred_element_type=jnp.float32)


def _tc3_body(p_ref, g_ref, nd_ref, b_ref, o_ref):
    o_ref[...] = (p_ref[0] + p_ref[1] + g_ref[...]) * nd_ref[...] + b_ref[...]


_vec_spec = pl.BlockSpec((RB, 1), lambda i: (i, 0))
_mat_spec = pl.BlockSpec((RB, D), lambda i: (i, 0))
_w_spec = pl.BlockSpec((D, D), lambda i: (0, 0))
_b_spec = pl.BlockSpec((1, D), lambda i: (0, 0))
_p_spec = pl.BlockSpec((2, RB, D), lambda i: (0, i, 0))

_tc1 = pl.pallas_call(
    _tc1_body,
    out_shape=jax.ShapeDtypeStruct((N, D), jnp.float32),
    grid=(N // RB,),
    in_specs=[_mat_spec, _vec_spec, _w_spec],
    out_specs=_mat_spec,
)

_tc2 = pl.pallas_call(
    _tc2_body,
    out_shape=jax.ShapeDtypeStruct((N, D), jnp.float32),
    grid=(N // RB,),
    in_specs=[_p_spec, _mat_spec, _vec_spec, _b_spec, _vec_spec, _w_spec],
    out_specs=_mat_spec,
)

_tc3 = pl.pallas_call(
    _tc3_body,
    out_shape=jax.ShapeDtypeStruct((N, D), jnp.float32),
    grid=(N // RB,),
    in_specs=[_p_spec, _mat_spec, _vec_spec, _b_spec],
    out_specs=_mat_spec,
)


def kernel(x, edge_index, W1, b1, W2, b2):
    src = edge_index[0]
    dst = edge_index[1]
    srcr = src.reshape(AGG_ROWS, CH)
    dstr = dst.reshape(AGG_ROWS, CH)
    idx2 = jnp.concatenate([src, dst + N]).reshape(DEG_ROWS, CH)

    upd = jnp.zeros((CH, DEG_W), jnp.float32).at[:, 0].set(1.0)
    zero16 = jnp.zeros((OC, DEG_W), jnp.float32)
    zero128 = jnp.zeros((CH, D), jnp.float32)

    degp = _deg_call(idx2, upd, zero16)
    deg = degp[: 2 * N, 0] + degp[2 * N :, 0] + 1.0
    ns = lax.rsqrt(deg[:N])[:, None]
    nd = lax.rsqrt(deg[N:])[:, None]

    g1 = _tc1(x, ns, W1)
    p1 = _agg_call(g1, srcr, dstr, zero128)
    g2 = _tc2(p1.reshape(2, N, D), g1, nd, b1.reshape(1, D), ns, W2)
    p2 = _agg_call(g2, srcr, dstr, zero128)
    out = _tc3(p2.reshape(2, N, D), g2, nd, b2.reshape(1, D))
    return out


# speedup vs baseline: 16.6434x; 1.0659x over previous
import jax
import jax.numpy as jnp
from jax import lax
from jax.experimental import pallas as pl
from jax.experimental.pallas import tpu as pltpu
from jax.experimental.pallas import tpu_sc as plsc

N = 10000
D = 128
E = 320000
NC = 2
NS = 16
NW = NC * NS

CH = 128

AGG_ROWS = E // CH
AGG_BASE = AGG_ROWS // NW
AGG_EXTRA = AGG_ROWS - AGG_BASE * NW

ROWS_PER_TILE = N // NS
DEG_W = 16

BLK = 26
NBLK = AGG_BASE // BLK


def _mesh():
    return plsc.VectorSubcoreMesh(core_axis_name="c", subcore_axis_name="s")


_sc_params = pltpu.CompilerParams(use_tc_tiling_on_sc=False)


def _deg_body(er_hbm, upd_hbm, zero_hbm, out_hbm,
              sidx, didx, ubuf, acc_s, acc_d, sem):
    c = lax.axis_index("c")
    s = lax.axis_index("s")
    w = c * NS + s

    pltpu.sync_copy(er_hbm.at[0, pl.ds(w * AGG_BASE, AGG_BASE)],
                    sidx.at[pl.ds(0, AGG_BASE)])
    pltpu.sync_copy(er_hbm.at[1, pl.ds(w * AGG_BASE, AGG_BASE)],
                    didx.at[pl.ds(0, AGG_BASE)])

    @pl.when(w < AGG_EXTRA)
    def _():
        pltpu.sync_copy(er_hbm.at[0, pl.ds(NW * AGG_BASE + w, 1)],
                        sidx.at[pl.ds(AGG_BASE, 1)])
        pltpu.sync_copy(er_hbm.at[1, pl.ds(NW * AGG_BASE + w, 1)],
                        didx.at[pl.ds(AGG_BASE, 1)])

    pltpu.sync_copy(upd_hbm, ubuf)

    pltpu.sync_copy(zero_hbm, acc_s.at[pl.ds(s * ROWS_PER_TILE, ROWS_PER_TILE)])
    pltpu.sync_copy(zero_hbm, acc_d.at[pl.ds(s * ROWS_PER_TILE, ROWS_PER_TILE)])
    plsc.subcore_barrier()

    @pl.loop(0, AGG_BASE // 3)
    def _(jj):
        j = jj * 3
        pltpu.async_copy(ubuf, acc_s.at[sidx.at[j]], sem, add=True)
        pltpu.async_copy(ubuf, acc_d.at[didx.at[j]], sem, add=True)
        pltpu.async_copy(ubuf, acc_s.at[sidx.at[j + 1]], sem, add=True)
        pltpu.async_copy(ubuf, acc_d.at[didx.at[j + 1]], sem, add=True)
        pltpu.async_copy(ubuf, acc_s.at[sidx.at[j + 2]], sem, add=True)
        pltpu.async_copy(ubuf, acc_d.at[didx.at[j + 2]], sem, add=True)
        pltpu.make_async_copy(ubuf, acc_s.at[sidx.at[j]], sem).wait()
        pltpu.make_async_copy(ubuf, acc_d.at[didx.at[j]], sem).wait()
        pltpu.make_async_copy(ubuf, acc_s.at[sidx.at[j + 1]], sem).wait()
        pltpu.make_async_copy(ubuf, acc_d.at[didx.at[j + 1]], sem).wait()
        pltpu.make_async_copy(ubuf, acc_s.at[sidx.at[j + 2]], sem).wait()
        pltpu.make_async_copy(ubuf, acc_d.at[didx.at[j + 2]], sem).wait()

    @pl.when(w < AGG_EXTRA)
    def _():
        pltpu.sync_copy(ubuf, acc_s.at[sidx.at[AGG_BASE]], add=True)
        pltpu.sync_copy(ubuf, acc_d.at[didx.at[AGG_BASE]], add=True)

    plsc.subcore_barrier()

    base = s * ROWS_PER_TILE
    pltpu.sync_copy(acc_s.at[pl.ds(base, ROWS_PER_TILE)],
                    out_hbm.at[pl.ds(c * 2 * N + base, ROWS_PER_TILE)])
    pltpu.sync_copy(acc_d.at[pl.ds(base, ROWS_PER_TILE)],
                    out_hbm.at[pl.ds(c * 2 * N + N + base, ROWS_PER_TILE)])


_deg_call = pl.kernel(
    _deg_body,
    out_type=jax.ShapeDtypeStruct((2 * 2 * N, DEG_W), jnp.float32),
    mesh=_mesh(),
    compiler_params=_sc_params,
    scratch_types=[
        pltpu.VMEM((AGG_BASE + 1, CH), jnp.int32),
        pltpu.VMEM((AGG_BASE + 1, CH), jnp.int32),
        pltpu.VMEM((CH, DEG_W), jnp.float32),
        pltpu.VMEM_SHARED((N, DEG_W), jnp.float32),
        pltpu.VMEM_SHARED((N, DEG_W), jnp.float32),
        pltpu.SemaphoreType.DMA,
    ],
)


def _agg_body(g_hbm, er_hbm, zero_hbm, out_hbm,
              sidx, didx, rows0, rows1, acc_sh, sem_g0, sem_g1, sem_i):
    c = lax.axis_index("c")
    s = lax.axis_index("s")
    w = c * NS + s

    pltpu.sync_copy(zero_hbm, acc_sh.at[pl.ds(s * ROWS_PER_TILE, ROWS_PER_TILE)])
    plsc.subcore_barrier()

    pltpu.sync_copy(er_hbm.at[0, pl.ds(w * AGG_BASE, BLK)], sidx.at[pl.ds(0, BLK)])
    pltpu.sync_copy(er_hbm.at[1, pl.ds(w * AGG_BASE, BLK)], didx.at[pl.ds(0, BLK)])
    pltpu.async_copy(g_hbm.at[sidx.at[0]], rows0, sem_g0)

    @pl.loop(0, NBLK)
    def _(kk):
        roff = (kk % 2) * BLK
        noff = ((kk + 1) % 2) * BLK

        @pl.when(kk + 1 < NBLK)
        def _():
            pltpu.async_copy(er_hbm.at[0, pl.ds(w * AGG_BASE + (kk + 1) * BLK, BLK)],
                             sidx.at[pl.ds(noff, BLK)], sem_i)
            pltpu.async_copy(er_hbm.at[1, pl.ds(w * AGG_BASE + (kk + 1) * BLK, BLK)],
                             didx.at[pl.ds(noff, BLK)], sem_i)

        @pl.loop(0, BLK // 2)
        def _(pp):
            j0 = roff + 2 * pp
            j1 = j0 + 1
            pltpu.make_async_copy(g_hbm.at[sidx.at[j0]], rows0, sem_g0).wait()
            pltpu.async_copy(g_hbm.at[sidx.at[j1]], rows1, sem_g1)
            pltpu.sync_copy(rows0, acc_sh.at[didx.at[j0]], add=True)

            @pl.when(2 * pp + 2 < BLK)
            def _():
                pltpu.async_copy(g_hbm.at[sidx.at[j0 + 2]], rows0, sem_g0)

            pltpu.make_async_copy(g_hbm.at[sidx.at[j1]], rows1, sem_g1).wait()
            pltpu.sync_copy(rows1, acc_sh.at[didx.at[j1]], add=True)

        @pl.when(kk + 1 < NBLK)
        def _():
            pltpu.make_async_copy(er_hbm.at[0, pl.ds(0, BLK)],
                                  sidx.at[pl.ds(noff, BLK)], sem_i).wait()
            pltpu.make_async_copy(er_hbm.at[1, pl.ds(0, BLK)],
                                  didx.at[pl.ds(noff, BLK)], sem_i).wait()
            pltpu.async_copy(g_hbm.at[sidx.at[noff]], rows0, sem_g0)

    @pl.when(w < AGG_EXTRA)
    def _():
        pltpu.sync_copy(er_hbm.at[0, pl.ds(NW * AGG_BASE + w, 1)],
                        sidx.at[pl.ds(0, 1)])
        pltpu.sync_copy(er_hbm.at[1, pl.ds(NW * AGG_BASE + w, 1)],
                        didx.at[pl.ds(0, 1)])
        pltpu.async_copy(g_hbm.at[sidx.at[0]], rows0, sem_g0).wait()
        pltpu.sync_copy(rows0, acc_sh.at[didx.at[0]], add=True)

    plsc.subcore_barrier()

    base = s * ROWS_PER_TILE
    pltpu.sync_copy(acc_sh.at[pl.ds(base, ROWS_PER_TILE)],
                    out_hbm.at[pl.ds(c * N + base, ROWS_PER_TILE)])


_agg_call = pl.kernel(
    _agg_body,
    out_type=jax.ShapeDtypeStruct((2 * N, D), jnp.float32),
    mesh=_mesh(),
    compiler_params=_sc_params,
    scratch_types=[
        pltpu.VMEM((2 * BLK, CH), jnp.int32),
        pltpu.VMEM((2 * BLK, CH), jnp.int32),
        pltpu.VMEM((CH, D), jnp.float32),
        pltpu.VMEM((CH, D), jnp.float32),
        pltpu.VMEM_SHARED((N, D), jnp.float32),
        pltpu.SemaphoreType.DMA,
        pltpu.SemaphoreType.DMA,
        pltpu.SemaphoreType.DMA,
    ],
)


RB = 1000


def _norm(a_ref, b_ref):
    return lax.rsqrt(a_ref[:, :1] + b_ref[:, :1] + 1.0)


def _tc1_body(x_ref, ds0_ref, ds1_ref, w_ref, o_ref):
    ns = _norm(ds0_ref, ds1_ref)
    o_ref[...] = jnp.dot(x_ref[...] * ns, w_ref[...],
                         preferred_element_type=jnp.float32)


def _tc2_body(p_ref, g_ref, dd0_ref, dd1_ref, ds0_ref, ds1_ref, b_ref, w_ref,
              o_ref):
    nd = _norm(dd0_ref, dd1_ref)
    ns = _norm(ds0_ref, ds1_ref)
    h = (p_ref[0] + p_ref[1] + g_ref[...]) * nd + b_ref[...]
    h = jnp.maximum(h, 0.0)
    o_ref[...] = jnp.dot(h * ns, w_ref[...], preferred_element_type=jnp.float32)


def _tc3_body(p_ref, g_ref, dd0_ref, dd1_ref, b_ref, o_ref):
    nd = _norm(dd0_ref, dd1_ref)
    o_ref[...] = (p_ref[0] + p_ref[1] + g_ref[...]) * nd + b_ref[...]


def _deg_spec(off):
    return pl.BlockSpec((RB, DEG_W), lambda i, off=off: (i + off, 0))


_NBLOCKS = N // RB
_S0 = 0
_D0 = _NBLOCKS
_S1 = 2 * _NBLOCKS
_D1 = 3 * _NBLOCKS

_mat_spec = pl.BlockSpec((RB, D), lambda i: (i, 0))
_w_spec = pl.BlockSpec((D, D), lambda i: (0, 0))
_b_spec = pl.BlockSpec((1, D), lambda i: (0, 0))
_p_spec = pl.BlockSpec((2, RB, D), lambda i: (0, i, 0))

_tc1 = pl.pallas_call(
    _tc1_body,
    out_shape=jax.ShapeDtypeStruct((N, D), jnp.float32),
    grid=(_NBLOCKS,),
    in_specs=[_mat_spec, _deg_spec(_S0), _deg_spec(_S1), _w_spec],
    out_specs=_mat_spec,
)

_tc2 = pl.pallas_call(
    _tc2_body,
    out_shape=jax.ShapeDtypeStruct((N, D), jnp.float32),
    grid=(_NBLOCKS,),
    in_specs=[_p_spec, _mat_spec, _deg_spec(_D0), _deg_spec(_D1),
              _deg_spec(_S0), _deg_spec(_S1), _b_spec, _w_spec],
    out_specs=_mat_spec,
)

_tc3 = pl.pallas_call(
    _tc3_body,
    out_shape=jax.ShapeDtypeStruct((N, D), jnp.float32),
    grid=(_NBLOCKS,),
    in_specs=[_p_spec, _mat_spec, _deg_spec(_D0), _deg_spec(_D1), _b_spec],
    out_specs=_mat_spec,
)


def kernel(x, edge_index, W1, b1, W2, b2):
    er = edge_index.reshape(2, AGG_ROWS, CH)
    upd = jnp.zeros((CH, DEG_W), jnp.float32).at[:, 0].set(1.0)
    zdeg = jnp.zeros((ROWS_PER_TILE, DEG_W), jnp.float32)
    zagg = jnp.zeros((ROWS_PER_TILE, D), jnp.float32)

    degp = _deg_call(er, upd, zdeg)
    g1 = _tc1(x, degp, degp, W1)
    p1 = _agg_call(g1, er, zagg)
    g2 = _tc2(p1.reshape(2, N, D), g1, degp, degp, degp, degp,
              b1.reshape(1, D), W2)
    p2 = _agg_call(g2, er, zagg)
    out = _tc3(p2.reshape(2, N, D), g2, degp, degp, b2.reshape(1, D))
    return out


# speedup vs baseline: 17.9383x; 1.0778x over previous
import jax
import jax.numpy as jnp
from jax import lax
from jax.experimental import pallas as pl
from jax.experimental.pallas import tpu as pltpu
from jax.experimental.pallas import tpu_sc as plsc

N = 10000
D = 128
E = 320000
NC = 2
NS = 16
NW = NC * NS

CH = 128

AGG_ROWS = E // CH
AGG_BASE = AGG_ROWS // NW
AGG_EXTRA = AGG_ROWS - AGG_BASE * NW

ROWS_PER_TILE = N // NS
DEG_W = 16

BLK = 26
NBLK = AGG_BASE // BLK


def _mesh():
    return plsc.VectorSubcoreMesh(core_axis_name="c", subcore_axis_name="s")


_sc_params = pltpu.CompilerParams(use_tc_tiling_on_sc=False)


def _deg_body(er_hbm, upd_hbm, zero_hbm, out_hbm,
              sidx, didx, ubuf, acc_s, acc_d, sem):
    c = lax.axis_index("c")
    s = lax.axis_index("s")
    w = c * NS + s

    pltpu.sync_copy(er_hbm.at[0, pl.ds(w * AGG_BASE, AGG_BASE)],
                    sidx.at[pl.ds(0, AGG_BASE)])
    pltpu.sync_copy(er_hbm.at[1, pl.ds(w * AGG_BASE, AGG_BASE)],
                    didx.at[pl.ds(0, AGG_BASE)])

    @pl.when(w < AGG_EXTRA)
    def _():
        pltpu.sync_copy(er_hbm.at[0, pl.ds(NW * AGG_BASE + w, 1)],
                        sidx.at[pl.ds(AGG_BASE, 1)])
        pltpu.sync_copy(er_hbm.at[1, pl.ds(NW * AGG_BASE + w, 1)],
                        didx.at[pl.ds(AGG_BASE, 1)])

    pltpu.sync_copy(upd_hbm, ubuf)

    pltpu.sync_copy(zero_hbm, acc_s.at[pl.ds(s * ROWS_PER_TILE, ROWS_PER_TILE)])
    pltpu.sync_copy(zero_hbm, acc_d.at[pl.ds(s * ROWS_PER_TILE, ROWS_PER_TILE)])
    plsc.subcore_barrier()

    @pl.loop(0, AGG_BASE // 3)
    def _(jj):
        j = jj * 3
        pltpu.async_copy(ubuf, acc_s.at[sidx.at[j]], sem, add=True)
        pltpu.async_copy(ubuf, acc_d.at[didx.at[j]], sem, add=True)
        pltpu.async_copy(ubuf, acc_s.at[sidx.at[j + 1]], sem, add=True)
        pltpu.async_copy(ubuf, acc_d.at[didx.at[j + 1]], sem, add=True)
        pltpu.async_copy(ubuf, acc_s.at[sidx.at[j + 2]], sem, add=True)
        pltpu.async_copy(ubuf, acc_d.at[didx.at[j + 2]], sem, add=True)
        pltpu.make_async_copy(ubuf, acc_s.at[sidx.at[j]], sem).wait()
        pltpu.make_async_copy(ubuf, acc_d.at[didx.at[j]], sem).wait()
        pltpu.make_async_copy(ubuf, acc_s.at[sidx.at[j + 1]], sem).wait()
        pltpu.make_async_copy(ubuf, acc_d.at[didx.at[j + 1]], sem).wait()
        pltpu.make_async_copy(ubuf, acc_s.at[sidx.at[j + 2]], sem).wait()
        pltpu.make_async_copy(ubuf, acc_d.at[didx.at[j + 2]], sem).wait()

    @pl.when(w < AGG_EXTRA)
    def _():
        pltpu.sync_copy(ubuf, acc_s.at[sidx.at[AGG_BASE]], add=True)
        pltpu.sync_copy(ubuf, acc_d.at[didx.at[AGG_BASE]], add=True)

    plsc.subcore_barrier()

    base = s * ROWS_PER_TILE
    pltpu.sync_copy(acc_s.at[pl.ds(base, ROWS_PER_TILE)],
                    out_hbm.at[pl.ds(c * 2 * N + base, ROWS_PER_TILE)])
    pltpu.sync_copy(acc_d.at[pl.ds(base, ROWS_PER_TILE)],
                    out_hbm.at[pl.ds(c * 2 * N + N + base, ROWS_PER_TILE)])


_deg_call = pl.kernel(
    _deg_body,
    out_type=jax.ShapeDtypeStruct((2 * 2 * N, DEG_W), jnp.float32),
    mesh=_mesh(),
    compiler_params=_sc_params,
    scratch_types=[
        pltpu.VMEM((AGG_BASE + 1, CH), jnp.int32),
        pltpu.VMEM((AGG_BASE + 1, CH), jnp.int32),
        pltpu.VMEM((CH, DEG_W), jnp.float32),
        pltpu.VMEM_SHARED((N, DEG_W), jnp.float32),
        pltpu.VMEM_SHARED((N, DEG_W), jnp.float32),
        pltpu.SemaphoreType.DMA,
    ],
)


def _agg_body(g_hbm, er_hbm, zero_hbm, out_hbm,
              sidx, didx, rows0, rows1, acc_sh, sem_g0, sem_g1, sem_i):
    c = lax.axis_index("c")
    s = lax.axis_index("s")
    w = c * NS + s

    pltpu.sync_copy(zero_hbm, acc_sh.at[pl.ds(s * ROWS_PER_TILE, ROWS_PER_TILE)])
    plsc.subcore_barrier()

    pltpu.sync_copy(er_hbm.at[0, pl.ds(w * AGG_BASE, BLK)], sidx.at[pl.ds(0, BLK)])
    pltpu.sync_copy(er_hbm.at[1, pl.ds(w * AGG_BASE, BLK)], didx.at[pl.ds(0, BLK)])
    pltpu.async_copy(g_hbm.at[sidx.at[0]], rows0, sem_g0)

    @pl.loop(0, NBLK)
    def _(kk):
        roff = (kk % 2) * BLK
        noff = ((kk + 1) % 2) * BLK

        @pl.when(kk + 1 < NBLK)
        def _():
            pltpu.async_copy(er_hbm.at[0, pl.ds(w * AGG_BASE + (kk + 1) * BLK, BLK)],
                             sidx.at[pl.ds(noff, BLK)], sem_i)
            pltpu.async_copy(er_hbm.at[1, pl.ds(w * AGG_BASE + (kk + 1) * BLK, BLK)],
                             didx.at[pl.ds(noff, BLK)], sem_i)

        @pl.loop(0, BLK // 2)
        def _(pp):
            j0 = roff + 2 * pp
            j1 = j0 + 1
            pltpu.make_async_copy(g_hbm.at[sidx.at[j0]], rows0, sem_g0).wait()
            pltpu.async_copy(g_hbm.at[sidx.at[j1]], rows1, sem_g1)
            pltpu.sync_copy(rows0, acc_sh.at[didx.at[j0]], add=True)

            @pl.when(2 * pp + 2 < BLK)
            def _():
                pltpu.async_copy(g_hbm.at[sidx.at[j0 + 2]], rows0, sem_g0)

            pltpu.make_async_copy(g_hbm.at[sidx.at[j1]], rows1, sem_g1).wait()
            pltpu.sync_copy(rows1, acc_sh.at[didx.at[j1]], add=True)

        @pl.when(kk + 1 < NBLK)
        def _():
            pltpu.make_async_copy(er_hbm.at[0, pl.ds(0, BLK)],
                                  sidx.at[pl.ds(noff, BLK)], sem_i).wait()
            pltpu.make_async_copy(er_hbm.at[1, pl.ds(0, BLK)],
                                  didx.at[pl.ds(noff, BLK)], sem_i).wait()
            pltpu.async_copy(g_hbm.at[sidx.at[noff]], rows0, sem_g0)

    @pl.when(w < AGG_EXTRA)
    def _():
        pltpu.sync_copy(er_hbm.at[0, pl.ds(NW * AGG_BASE + w, 1)],
                        sidx.at[pl.ds(0, 1)])
        pltpu.sync_copy(er_hbm.at[1, pl.ds(NW * AGG_BASE + w, 1)],
                        didx.at[pl.ds(0, 1)])
        pltpu.async_copy(g_hbm.at[sidx.at[0]], rows0, sem_g0).wait()
        pltpu.sync_copy(rows0, acc_sh.at[didx.at[0]], add=True)

    plsc.subcore_barrier()

    base = s * ROWS_PER_TILE
    pltpu.sync_copy(acc_sh.at[pl.ds(base, ROWS_PER_TILE)],
                    out_hbm.at[pl.ds(c * N + base, ROWS_PER_TILE)])


_agg_call = pl.kernel(
    _agg_body,
    out_type=jax.ShapeDtypeStruct((2 * N, D), jnp.bfloat16),
    mesh=_mesh(),
    compiler_params=_sc_params,
    scratch_types=[
        pltpu.VMEM((2 * BLK, CH), jnp.int32),
        pltpu.VMEM((2 * BLK, CH), jnp.int32),
        pltpu.VMEM((CH, D), jnp.bfloat16),
        pltpu.VMEM((CH, D), jnp.bfloat16),
        pltpu.VMEM_SHARED((N, D), jnp.bfloat16),
        pltpu.SemaphoreType.DMA,
        pltpu.SemaphoreType.DMA,
        pltpu.SemaphoreType.DMA,
    ],
)


RB = 1000


def _norm(a_ref, b_ref):
    return lax.rsqrt(a_ref[:, :1] + b_ref[:, :1] + 1.0)


def _tc1_body(x_ref, ds0_ref, ds1_ref, w_ref, o_ref):
    ns = _norm(ds0_ref, ds1_ref)
    o_ref[...] = jnp.dot(x_ref[...] * ns, w_ref[...],
                         preferred_element_type=jnp.float32).astype(jnp.bfloat16)


def _tc2_body(p_ref, g_ref, dd0_ref, dd1_ref, ds0_ref, ds1_ref, b_ref, w_ref,
              o_ref):
    nd = _norm(dd0_ref, dd1_ref)
    ns = _norm(ds0_ref, ds1_ref)
    h = ((p_ref[0] + p_ref[1]).astype(jnp.float32)
         + g_ref[...].astype(jnp.float32)) * nd + b_ref[...]
    h = jnp.maximum(h, 0.0)
    o_ref[...] = jnp.dot(h * ns, w_ref[...],
                         preferred_element_type=jnp.float32).astype(jnp.bfloat16)


def _tc3_body(p_ref, g_ref, dd0_ref, dd1_ref, b_ref, o_ref):
    nd = _norm(dd0_ref, dd1_ref)
    o_ref[...] = ((p_ref[0] + p_ref[1]).astype(jnp.float32)
                  + g_ref[...].astype(jnp.float32)) * nd + b_ref[...]


def _deg_spec(off):
    return pl.BlockSpec((RB, DEG_W), lambda i, off=off: (i + off, 0))


_NBLOCKS = N // RB
_S0 = 0
_D0 = _NBLOCKS
_S1 = 2 * _NBLOCKS
_D1 = 3 * _NBLOCKS

_mat_spec = pl.BlockSpec((RB, D), lambda i: (i, 0))
_w_spec = pl.BlockSpec((D, D), lambda i: (0, 0))
_b_spec = pl.BlockSpec((1, D), lambda i: (0, 0))
_p_spec = pl.BlockSpec((2, RB, D), lambda i: (0, i, 0))

_tc1 = pl.pallas_call(
    _tc1_body,
    out_shape=jax.ShapeDtypeStruct((N, D), jnp.bfloat16),
    grid=(_NBLOCKS,),
    in_specs=[_mat_spec, _deg_spec(_S0), _deg_spec(_S1), _w_spec],
    out_specs=_mat_spec,
)

_tc2 = pl.pallas_call(
    _tc2_body,
    out_shape=jax.ShapeDtypeStruct((N, D), jnp.bfloat16),
    grid=(_NBLOCKS,),
    in_specs=[_p_spec, _mat_spec, _deg_spec(_D0), _deg_spec(_D1),
              _deg_spec(_S0), _deg_spec(_S1), _b_spec, _w_spec],
    out_specs=_mat_spec,
)

_tc3 = pl.pallas_call(
    _tc3_body,
    out_shape=jax.ShapeDtypeStruct((N, D), jnp.float32),
    grid=(_NBLOCKS,),
    in_specs=[_p_spec, _mat_spec, _deg_spec(_D0), _deg_spec(_D1), _b_spec],
    out_specs=_mat_spec,
)


def kernel(x, edge_index, W1, b1, W2, b2):
    er = edge_index.reshape(2, AGG_ROWS, CH)
    upd = jnp.zeros((CH, DEG_W), jnp.float32).at[:, 0].set(1.0)
    zdeg = jnp.zeros((ROWS_PER_TILE, DEG_W), jnp.float32)
    zagg = jnp.zeros((ROWS_PER_TILE, D), jnp.bfloat16)

    degp = _deg_call(er, upd, zdeg)
    g1 = _tc1(x, degp, degp, W1)
    p1 = _agg_call(g1, er, zagg)
    g2 = _tc2(p1.reshape(2, N, D), g1, degp, degp, degp, degp,
              b1.reshape(1, D), W2)
    p2 = _agg_call(g2, er, zagg)
    out = _tc3(p2.reshape(2, N, D), g2, degp, degp, b2.reshape(1, D))
    return out
